# Initial kernel scaffold; baseline (speedup 1.0000x reference)
import jax
import jax.numpy as jnp
from jax.experimental import pallas as pl


def kernel(token_ids, E):
    raise NotImplementedError("write your pallas kernel here")


# speedup vs baseline: 1.2857x; 1.2857x over previous
import functools

import jax
import jax.numpy as jnp
from jax import lax
from jax.experimental import pallas as pl
from jax.experimental.pallas import tpu as pltpu
from jax.experimental.pallas import tpu_sc as plsc

_NC, _NS = 2, 16
_NW = _NC * _NS
_IW = 128
_G = 8


def _sc_gather(idx2d, table):
    R = idx2d.shape[0]
    D = table.shape[1]
    rpw = R // _NW
    mesh = plsc.VectorSubcoreMesh(
        core_axis_name="c", subcore_axis_name="s",
        num_cores=_NC, num_subcores=_NS)

    @functools.partial(
        pl.kernel,
        out_type=jax.ShapeDtypeStruct((R, _IW, D), jnp.float32),
        mesh=mesh,
        scratch_types=[
            pltpu.VMEM((_G, _IW), jnp.int32),
            pltpu.VMEM((_G, _IW, D), jnp.float32),
            pltpu.SemaphoreType.DMA,
        ],
        compiler_params=pltpu.CompilerParams(use_tc_tiling_on_sc=False),
    )
    def k(idx_hbm, tab_hbm, out_hbm, idx_v, rows_v, sem):
        w = lax.axis_index("s") * _NC + lax.axis_index("c")
        base0 = w * rpw

        @pl.loop(0, rpw // _G)
        def _group(i):
            base = base0 + i * _G
            pltpu.sync_copy(idx_hbm.at[pl.ds(base, _G)], idx_v)
            copies = [
                pltpu.async_copy(tab_hbm.at[idx_v.at[j]], rows_v.at[j], sem)
                for j in range(_G)
            ]
            for c in copies:
                c.wait()
            pltpu.sync_copy(rows_v, out_hbm.at[pl.ds(base, _G)])

    return k(idx2d, table)


def kernel(token_ids, E):
    B0, B1 = token_ids.shape
    D = E.shape[1]
    idx = token_ids.reshape(-1, _IW).astype(jnp.int32)
    out = _sc_gather(idx, E)
    return out.reshape(B0, B1, D)


# speedup vs baseline: 1.3024x; 1.0130x over previous
import functools

import jax
import jax.numpy as jnp
from jax import lax
from jax.experimental import pallas as pl
from jax.experimental.pallas import tpu as pltpu
from jax.experimental.pallas import tpu_sc as plsc

_NC, _NS = 2, 16
_NW = _NC * _NS
_IW = 128
_G = 10


def _sc_gather(idx2d, table):
    R = idx2d.shape[0]
    D = table.shape[1]
    rpw = R // _NW
    n_groups = rpw // _G
    mesh = plsc.VectorSubcoreMesh(
        core_axis_name="c", subcore_axis_name="s",
        num_cores=_NC, num_subcores=_NS)

    @functools.partial(
        pl.kernel,
        out_type=jax.ShapeDtypeStruct((R, _IW, D), jnp.float32),
        mesh=mesh,
        scratch_types=[
            pltpu.VMEM((2, _G, _IW), jnp.int32),
            pltpu.VMEM((2, _G, _IW, D), jnp.float32),
            pltpu.SemaphoreType.DMA,
            pltpu.SemaphoreType.DMA,
            pltpu.SemaphoreType.DMA,
        ],
        compiler_params=pltpu.CompilerParams(use_tc_tiling_on_sc=False),
    )
    def k(idx_hbm, tab_hbm, out_hbm, idx_v, rows_v, gsem, wsem0, wsem1):
        w = lax.axis_index("s") * _NC + lax.axis_index("c")
        base0 = w * rpw
        wsems = (wsem0, wsem1)

        def do_group(g, b, wait_writeback):
            base = base0 + g * _G
            pltpu.sync_copy(idx_hbm.at[pl.ds(base, _G)], idx_v.at[b])
            if wait_writeback:
                pltpu.make_async_copy(
                    rows_v.at[b], out_hbm.at[pl.ds(base, _G)], wsems[b]).wait()
            copies = [
                pltpu.async_copy(
                    tab_hbm.at[idx_v.at[b].at[j]], rows_v.at[b].at[j], gsem)
                for j in range(_G)
            ]
            for c in copies:
                c.wait()
            pltpu.async_copy(rows_v.at[b], out_hbm.at[pl.ds(base, _G)], wsems[b])

        do_group(0, 0, False)
        do_group(1, 1, False)

        @pl.loop(2, n_groups, step=2)
        def _pair(g):
            do_group(g, 0, True)
            do_group(g + 1, 1, True)

        for b in range(2):
            tail = base0 + (n_groups - 2 + b) * _G
            pltpu.make_async_copy(
                rows_v.at[b], out_hbm.at[pl.ds(tail, _G)], wsems[b]).wait()

    return k(idx2d, table)


def kernel(token_ids, E):
    B0, B1 = token_ids.shape
    D = E.shape[1]
    idx = token_ids.reshape(-1, _IW).astype(jnp.int32)
    out = _sc_gather(idx, E)
    return out.reshape(B0, B1, D)


# speedup vs baseline: 1.3916x; 1.0685x over previous
import functools

import jax
import jax.numpy as jnp
from jax import lax
from jax.experimental import pallas as pl
from jax.experimental.pallas import tpu as pltpu
from jax.experimental.pallas import tpu_sc as plsc

_NC, _NS = 2, 16
_NW = _NC * _NS
_IW = 128
_GPB = 4
_BLK = _GPB * _IW
_L = 16


def _sc_gather(tt3, table):
    S = tt3.shape[0]
    B = _NW * _BLK
    D = table.shape[1]
    mesh = plsc.VectorSubcoreMesh(
        core_axis_name="c", subcore_axis_name="s",
        num_cores=_NC, num_subcores=_NS)

    @functools.partial(
        pl.kernel,
        out_type=jax.ShapeDtypeStruct((S, D, B), jnp.float32),
        mesh=mesh,
        scratch_types=[
            pltpu.VMEM((2, _GPB, _IW), jnp.int32),
            pltpu.VMEM((2, _BLK, D), jnp.float32),
            pltpu.VMEM((2, D, _BLK), jnp.float32),
            pltpu.SemaphoreType.DMA,
            pltpu.SemaphoreType.DMA,
            pltpu.SemaphoreType.DMA,
        ],
        compiler_params=pltpu.CompilerParams(
            use_tc_tiling_on_sc=False, needs_layout_passes=False),
    )
    def k(idx_hbm, tab_hbm, out_hbm, idx_v, rows_v, rowsT_v, gsem, wsem0, wsem1):
        w = lax.axis_index("s") * _NC + lax.axis_index("c")
        b0 = w * _BLK
        wsems = (wsem0, wsem1)
        lane = lax.iota(jnp.int32, _L)

        def do_plane(s, b, wait_writeback):
            pltpu.sync_copy(idx_hbm.at[s, pl.ds(w * _GPB, _GPB)], idx_v.at[b])
            if wait_writeback:
                pltpu.make_async_copy(
                    rowsT_v.at[b], out_hbm.at[s, :, pl.ds(b0, _BLK)],
                    wsems[b]).wait()
            copies = [
                pltpu.async_copy(
                    tab_hbm.at[idx_v.at[b].at[j]],
                    rows_v.at[b, pl.ds(j * _IW, _IW)], gsem)
                for j in range(_GPB)
            ]
            for c in copies:
                c.wait()

            @pl.loop(0, _BLK // _L)
            def _tg(g):
                row_ids = g * _L + lane
                for d in range(D):
                    v = plsc.load_gather(
                        rows_v.at[b], [row_ids, jnp.full((_L,), d, jnp.int32)])
                    rowsT_v[b, d, pl.ds(g * _L, _L)] = v

            pltpu.async_copy(
                rowsT_v.at[b], out_hbm.at[s, :, pl.ds(b0, _BLK)], wsems[b])

        do_plane(0, 0, False)
        do_plane(1, 1, False)

        @pl.loop(2, S, step=2)
        def _pair(s):
            do_plane(s, 0, True)
            do_plane(s + 1, 1, True)

        for b in range(2):
            pltpu.make_async_copy(
                rowsT_v.at[b], out_hbm.at[S - 2 + b, :, pl.ds(b0, _BLK)],
                wsems[b]).wait()

    return k(tt3, table)


def kernel(token_ids, E):
    B0, B1 = token_ids.shape
    D = E.shape[1]
    tt3 = token_ids.T.reshape(B1, B0 // _IW, _IW).astype(jnp.int32)
    out = _sc_gather(tt3, E)
    return jnp.transpose(out, (2, 0, 1))
